# Initial kernel scaffold; baseline (speedup 1.0000x reference)
import jax
import jax.numpy as jnp
from jax.experimental import pallas as pl


def kernel(x, edge_index, edge_features, Wn, bn, We, be, Wl0, bl0, Wr0, br0, att0, bg0, Wl1, bl1, Wr1, br1, att1, bg1, Wl2, bl2, Wr2, br2, att2, bg2, W1, b1, W2, b2, W3, b3):
    raise NotImplementedError("write your pallas kernel here")


# speedup vs baseline: 5.4514x; 5.4514x over previous
import functools

import jax
import jax.numpy as jnp
from jax.experimental import pallas as pl

N = 10000
E = 160000
H = 4
C = 64
HC = H * C
EB = 2000


def _sel_M():
    jj = jax.lax.broadcasted_iota(jnp.int32, (HC, H), 0)
    hh = jax.lax.broadcasted_iota(jnp.int32, (HC, H), 1)
    return (jj // C == hh).astype(jnp.float32)


def _sel_K():
    hh = jax.lax.broadcasted_iota(jnp.int32, (H, HC), 0)
    jj = jax.lax.broadcasted_iota(jnp.int32, (H, HC), 1)
    return (jj // C == hh).astype(jnp.float32)


def _sel_P():
    jj = jax.lax.broadcasted_iota(jnp.int32, (HC, C), 0)
    cc = jax.lax.broadcasted_iota(jnp.int32, (HC, C), 1)
    return ((jj % C == cc).astype(jnp.float32)) * (1.0 / H)


def _node_init_kernel(x_ref, wn_ref, bn_ref, h_ref):
    h_ref[...] = jnp.dot(x_ref[...], wn_ref[...],
                         preferred_element_type=jnp.float32) + bn_ref[...]


def _lr_proj_kernel(h_ref, wl_ref, bl_ref, wr_ref, br_ref, xl_ref, xr_ref):
    h = h_ref[...]
    xl_ref[...] = jnp.dot(h, wl_ref[...],
                          preferred_element_type=jnp.float32) + bl_ref[...]
    xr_ref[...] = jnp.dot(h, wr_ref[...],
                          preferred_element_type=jnp.float32) + br_ref[...]


def _edge_score_kernel(xls_ref, xrd_ref, att_ref, ee_ref):
    g = xls_ref[...] + xrd_ref[...]
    g = jnp.where(g > 0, g, 0.2 * g)
    t = g * att_ref[...]
    e = jnp.dot(t, _sel_M(), preferred_element_type=jnp.float32)
    ee_ref[...] = jnp.exp(e)


def _edge_weight_kernel(xls_ref, ee_ref, dend_ref, w_ref):
    a = ee_ref[...] / dend_ref[...]
    ar = jnp.dot(a, _sel_K(), preferred_element_type=jnp.float32)
    w_ref[...] = xls_ref[...] * ar


def _head_mean_kernel(apply_elu, agg_ref, bg_ref, h_ref):
    h = jnp.dot(agg_ref[...], _sel_P(),
                preferred_element_type=jnp.float32) + bg_ref[...]
    if apply_elu:
        h = jnp.where(h > 0, h, jnp.exp(h) - 1.0)
    h_ref[...] = h


def _edge_mlp_kernel(hs_ref, hd_ref, ef_ref, we_ref, be_ref,
                     w1a_ref, w1b_ref, w1c_ref, b1_ref,
                     w2_ref, b2_ref, w3_ref, b3_ref, out_ref):
    emb = jnp.dot(ef_ref[...], we_ref[...],
                  preferred_element_type=jnp.float32) + be_ref[...]
    z = (jnp.dot(hs_ref[...], w1a_ref[...], preferred_element_type=jnp.float32)
         + jnp.dot(hd_ref[...], w1b_ref[...], preferred_element_type=jnp.float32)
         + jnp.dot(emb, w1c_ref[...], preferred_element_type=jnp.float32)
         + b1_ref[...])
    z = jnp.maximum(z, 0.0)
    z = jnp.dot(z, w2_ref[...], preferred_element_type=jnp.float32) + b2_ref[...]
    z = jnp.maximum(z, 0.0)
    z = jnp.dot(z, w3_ref[...], preferred_element_type=jnp.float32) + b3_ref[...]
    out_ref[...] = jnp.maximum(z, 0.0)


def _full(shape):
    return pl.BlockSpec(shape, lambda i: (0, 0))


def _rows(shape):
    return pl.BlockSpec(shape, lambda i: (i, 0))


def _gat_layer(h, src, dst, et, Wl, bl, Wr, br, attf, bg, apply_elu):
    xl, xr = pl.pallas_call(
        _lr_proj_kernel,
        out_shape=(jax.ShapeDtypeStruct((N, HC), jnp.float32),
                   jax.ShapeDtypeStruct((N, HC), jnp.float32)),
    )(h, Wl, bl, Wr, br)

    xls = jnp.take(xl, src, axis=0)
    xrd = jnp.take(xr, dst, axis=0)

    grid = (et // EB,)
    ee = pl.pallas_call(
        _edge_score_kernel,
        grid=grid,
        in_specs=[_rows((EB, HC)), _rows((EB, HC)), _full((1, HC))],
        out_specs=_rows((EB, H)),
        out_shape=jax.ShapeDtypeStruct((et, H), jnp.float32),
    )(xls, xrd, attf)

    den = jax.ops.segment_sum(ee, dst, num_segments=N)
    dend = jnp.take(den, dst, axis=0)

    w = pl.pallas_call(
        _edge_weight_kernel,
        grid=grid,
        in_specs=[_rows((EB, HC)), _rows((EB, H)), _rows((EB, H))],
        out_specs=_rows((EB, HC)),
        out_shape=jax.ShapeDtypeStruct((et, HC), jnp.float32),
    )(xls, ee, dend)

    agg = jax.ops.segment_sum(w, dst, num_segments=N)

    return pl.pallas_call(
        functools.partial(_head_mean_kernel, apply_elu),
        out_shape=jax.ShapeDtypeStruct((N, C), jnp.float32),
    )(agg, bg)


def kernel(x, edge_index, edge_features, Wn, bn, We, be,
           Wl0, bl0, Wr0, br0, att0, bg0,
           Wl1, bl1, Wr1, br1, att1, bg1,
           Wl2, bl2, Wr2, br2, att2, bg2,
           W1, b1, W2, b2, W3, b3):
    loops = jnp.arange(N, dtype=edge_index.dtype)
    src = jnp.concatenate([edge_index[0], loops])
    dst = jnp.concatenate([edge_index[1], loops])
    et = E + N

    h = pl.pallas_call(
        _node_init_kernel,
        out_shape=jax.ShapeDtypeStruct((N, C), jnp.float32),
    )(x, Wn, bn.reshape(1, C))

    layers = [
        (Wl0, bl0, Wr0, br0, att0, bg0, True),
        (Wl1, bl1, Wr1, br1, att1, bg1, True),
        (Wl2, bl2, Wr2, br2, att2, bg2, False),
    ]
    for Wl, bl, Wr, br, att, bg, elu in layers:
        h = _gat_layer(h, src, dst, et, Wl, bl.reshape(1, HC),
                       Wr, br.reshape(1, HC), att.reshape(1, HC),
                       bg.reshape(1, C), elu)

    hs = jnp.take(h, edge_index[0], axis=0)
    hd = jnp.take(h, edge_index[1], axis=0)

    W1a, W1b, W1c = W1[:C], W1[C:2 * C], W1[2 * C:]
    out = pl.pallas_call(
        _edge_mlp_kernel,
        grid=(E // EB,),
        in_specs=[_rows((EB, C)), _rows((EB, C)), _rows((EB, 3)),
                  _full((3, C)), _full((1, C)),
                  _full((C, C)), _full((C, C)), _full((C, C)), _full((1, C)),
                  _full((C, C // 2)), _full((1, C // 2)),
                  _full((C // 2, 1)), _full((1, 1))],
        out_specs=_rows((EB, 1)),
        out_shape=jax.ShapeDtypeStruct((E, 1), jnp.float32),
    )(hs, hd, edge_features, We, be.reshape(1, C),
      W1a, W1b, W1c, b1.reshape(1, C),
      W2, b2.reshape(1, C // 2), W3, b3.reshape(1, 1))
    return out


# speedup vs baseline: 5.4825x; 1.0057x over previous
import functools

import jax
import jax.numpy as jnp
from jax.experimental import pallas as pl

N = 10000
E = 160000
H = 4
C = 64
HC = H * C
EB = 10000


def _sel_M():
    jj = jax.lax.broadcasted_iota(jnp.int32, (HC, H), 0)
    hh = jax.lax.broadcasted_iota(jnp.int32, (HC, H), 1)
    return (jj // C == hh).astype(jnp.float32)


def _sel_K():
    hh = jax.lax.broadcasted_iota(jnp.int32, (H, HC), 0)
    jj = jax.lax.broadcasted_iota(jnp.int32, (H, HC), 1)
    return (jj // C == hh).astype(jnp.float32)


def _sel_P():
    jj = jax.lax.broadcasted_iota(jnp.int32, (HC, C), 0)
    cc = jax.lax.broadcasted_iota(jnp.int32, (HC, C), 1)
    return ((jj % C == cc).astype(jnp.float32)) * (1.0 / H)


def _node_init_kernel(x_ref, wn_ref, bn_ref, h_ref):
    h_ref[...] = jnp.dot(x_ref[...], wn_ref[...],
                         preferred_element_type=jnp.float32) + bn_ref[...]


def _lr_proj_kernel(h_ref, wl_ref, bl_ref, wr_ref, br_ref, xl_ref, xr_ref):
    h = h_ref[...]
    xl_ref[...] = jnp.dot(h, wl_ref[...],
                          preferred_element_type=jnp.float32) + bl_ref[...]
    xr_ref[...] = jnp.dot(h, wr_ref[...],
                          preferred_element_type=jnp.float32) + br_ref[...]


def _edge_score_kernel(xls_ref, xrd_ref, att_ref, ee_ref):
    g = xls_ref[...] + xrd_ref[...]
    g = jnp.where(g > 0, g, 0.2 * g)
    t = g * att_ref[...]
    e = jnp.dot(t, _sel_M(), preferred_element_type=jnp.float32)
    ee_ref[...] = jnp.exp(e)


def _edge_weight_kernel(xls_ref, ee_ref, dend_ref, w_ref):
    a = ee_ref[...] / dend_ref[...]
    ar = jnp.dot(a, _sel_K(), preferred_element_type=jnp.float32)
    w_ref[...] = xls_ref[...] * ar


def _head_mean_kernel(apply_elu, agg_ref, bg_ref, h_ref):
    h = jnp.dot(agg_ref[...], _sel_P(),
                preferred_element_type=jnp.float32) + bg_ref[...]
    if apply_elu:
        h = jnp.where(h > 0, h, jnp.exp(h) - 1.0)
    h_ref[...] = h


def _edge_mlp_kernel(hs_ref, hd_ref, ef_ref, we_ref, be_ref,
                     w1a_ref, w1b_ref, w1c_ref, b1_ref,
                     w2_ref, b2_ref, w3_ref, b3_ref, out_ref):
    emb = jnp.dot(ef_ref[...], we_ref[...],
                  preferred_element_type=jnp.float32) + be_ref[...]
    z = (jnp.dot(hs_ref[...], w1a_ref[...], preferred_element_type=jnp.float32)
         + jnp.dot(hd_ref[...], w1b_ref[...], preferred_element_type=jnp.float32)
         + jnp.dot(emb, w1c_ref[...], preferred_element_type=jnp.float32)
         + b1_ref[...])
    z = jnp.maximum(z, 0.0)
    z = jnp.dot(z, w2_ref[...], preferred_element_type=jnp.float32) + b2_ref[...]
    z = jnp.maximum(z, 0.0)
    z = jnp.dot(z, w3_ref[...], preferred_element_type=jnp.float32) + b3_ref[...]
    out_ref[...] = jnp.maximum(z, 0.0)


def _full(shape):
    return pl.BlockSpec(shape, lambda i: (0, 0))


def _rows(shape):
    return pl.BlockSpec(shape, lambda i: (i, 0))


def _gat_layer(h, src, dst, et, Wl, bl, Wr, br, attf, bg, apply_elu):
    xl, xr = pl.pallas_call(
        _lr_proj_kernel,
        out_shape=(jax.ShapeDtypeStruct((N, HC), jnp.float32),
                   jax.ShapeDtypeStruct((N, HC), jnp.float32)),
    )(h, Wl, bl, Wr, br)

    xls = jnp.take(xl, src, axis=0)
    xrd = jnp.take(xr, dst, axis=0)

    grid = (et // EB,)
    ee = pl.pallas_call(
        _edge_score_kernel,
        grid=grid,
        in_specs=[_rows((EB, HC)), _rows((EB, HC)), _full((1, HC))],
        out_specs=_rows((EB, H)),
        out_shape=jax.ShapeDtypeStruct((et, H), jnp.float32),
    )(xls, xrd, attf)

    den = jax.ops.segment_sum(ee, dst, num_segments=N)
    dend = jnp.take(den, dst, axis=0)

    w = pl.pallas_call(
        _edge_weight_kernel,
        grid=grid,
        in_specs=[_rows((EB, HC)), _rows((EB, H)), _rows((EB, H))],
        out_specs=_rows((EB, HC)),
        out_shape=jax.ShapeDtypeStruct((et, HC), jnp.float32),
    )(xls, ee, dend)

    agg = jax.ops.segment_sum(w, dst, num_segments=N)

    return pl.pallas_call(
        functools.partial(_head_mean_kernel, apply_elu),
        out_shape=jax.ShapeDtypeStruct((N, C), jnp.float32),
    )(agg, bg)


def kernel(x, edge_index, edge_features, Wn, bn, We, be,
           Wl0, bl0, Wr0, br0, att0, bg0,
           Wl1, bl1, Wr1, br1, att1, bg1,
           Wl2, bl2, Wr2, br2, att2, bg2,
           W1, b1, W2, b2, W3, b3):
    loops = jnp.arange(N, dtype=edge_index.dtype)
    src = jnp.concatenate([edge_index[0], loops])
    dst = jnp.concatenate([edge_index[1], loops])
    et = E + N

    h = pl.pallas_call(
        _node_init_kernel,
        out_shape=jax.ShapeDtypeStruct((N, C), jnp.float32),
    )(x, Wn, bn.reshape(1, C))

    layers = [
        (Wl0, bl0, Wr0, br0, att0, bg0, True),
        (Wl1, bl1, Wr1, br1, att1, bg1, True),
        (Wl2, bl2, Wr2, br2, att2, bg2, False),
    ]
    for Wl, bl, Wr, br, att, bg, elu in layers:
        h = _gat_layer(h, src, dst, et, Wl, bl.reshape(1, HC),
                       Wr, br.reshape(1, HC), att.reshape(1, HC),
                       bg.reshape(1, C), elu)

    hs = jnp.take(h, edge_index[0], axis=0)
    hd = jnp.take(h, edge_index[1], axis=0)

    W1a, W1b, W1c = W1[:C], W1[C:2 * C], W1[2 * C:]
    out = pl.pallas_call(
        _edge_mlp_kernel,
        grid=(E // EB,),
        in_specs=[_rows((EB, C)), _rows((EB, C)), _rows((EB, 3)),
                  _full((3, C)), _full((1, C)),
                  _full((C, C)), _full((C, C)), _full((C, C)), _full((1, C)),
                  _full((C, C // 2)), _full((1, C // 2)),
                  _full((C // 2, 1)), _full((1, 1))],
        out_specs=_rows((EB, 1)),
        out_shape=jax.ShapeDtypeStruct((E, 1), jnp.float32),
    )(hs, hd, edge_features, We, be.reshape(1, C),
      W1a, W1b, W1c, b1.reshape(1, C),
      W2, b2.reshape(1, C // 2), W3, b3.reshape(1, 1))
    return out
